# Initial kernel scaffold; baseline (speedup 1.0000x reference)
import functools

import jax
import jax.numpy as jnp
from jax import lax
from jax.experimental import pallas as pl
from jax.experimental.pallas import tpu as pltpu
from jax.experimental.pallas import tpu_sc as plsc

N = 10000
E = 160000
MAX_NB = 8
IN_DIM = 16
FEAT = 128
OUT = 128

NC, NS = 2, 16
NW = NC * NS
LANES = 16

N_PAD = 10240


def _make_gsum_f32(M, D, B):
    Mw = M // NW
    NB = Mw // B
    K = B * MAX_NB
    mesh = plsc.VectorSubcoreMesh(core_axis_name="c", subcore_axis_name="s")

    def body(table_hbm, idx_hbm, out_hbm, idx_v, rows_v, out_v, sem):
        wid = lax.axis_index("s") * NC + lax.axis_index("c")
        row0 = wid * Mw
        pltpu.sync_copy(idx_hbm.at[pl.ds(row0 * MAX_NB, Mw * MAX_NB)], idx_v)

        def block(b, carry):
            base = b * K
            pltpu.async_copy(
                table_hbm.at[idx_v.at[pl.ds(base, K)]], rows_v, sem).wait()
            for i in range(B):
                for c in range(D // LANES):
                    sl = pl.ds(c * LANES, LANES)
                    acc = rows_v[MAX_NB * i, sl]
                    for j in range(1, MAX_NB):
                        acc = acc + rows_v[MAX_NB * i + j, sl]
                    out_v[i, sl] = acc
            pltpu.sync_copy(out_v, out_hbm.at[pl.ds(row0 + b * B, B)])
            return carry

        lax.fori_loop(0, NB, block, 0)

    return pl.kernel(
        body,
        out_type=jax.ShapeDtypeStruct((M, D), jnp.float32),
        mesh=mesh,
        scratch_types=[
            pltpu.VMEM((Mw * MAX_NB,), jnp.int32),
            pltpu.VMEM((K, D), jnp.float32),
            pltpu.VMEM((B, D), jnp.float32),
            pltpu.SemaphoreType.DMA,
        ],
    )


def _dot(a, w):
    return jnp.dot(a, w, preferred_element_type=jnp.float32)


_RB_E = 2000
_RB_N = 2000


def _k1_body(ef, wi, bi, h0):
    h0[...] = jnp.maximum(_dot(ef[...], wi[...]) + bi[...], 0.0)


def _k2_body(h0, nei, wh, bh, h1):
    h1[...] = jnp.maximum(h0[...] + _dot(nei[...], wh[...]) + bh[...], 0.0)


def _k3_body(h0, nei, wh, bh, wp, bp, eh):
    t = jnp.maximum(h0[...] + _dot(nei[...], wh[...]) + bh[...], 0.0)
    eh[...] = _dot(t, wp[...]) + bp[...]


def _k4_body(nf, nm, wt1, wt2, bt, out):
    y = jnp.maximum(_dot(nf[...], wt1[...]) + _dot(nm[...], wt2[...]) + bt[...], 0.0)
    rows = lax.broadcasted_iota(jnp.int32, y.shape, 0)
    out[...] = jnp.where((pl.program_id(0) == 0) & (rows == 0), 0.0, y)


def _row_spec(rb, d):
    return pl.BlockSpec((rb, d), lambda i: (i, 0))


def _full_spec(r, c):
    return pl.BlockSpec((r, c), lambda i: (0, 0))


def kernel(node_features, edge_features, adj_list, bond_list,
           W_i, b_i, W_h, b_h, W_proj, b_proj, W_t, b_t):
    bond_flat = bond_list.reshape(-1)
    adj_flat = jnp.zeros((N_PAD * MAX_NB,), jnp.int32).at[: N * MAX_NB].set(
        adj_list.reshape(-1))
    bi = b_i.reshape(1, OUT)
    bh = b_h.reshape(1, OUT)
    bp = b_proj.reshape(1, OUT)
    bt = b_t.reshape(1, OUT)
    wt1 = W_t[:FEAT]
    wt2 = W_t[FEAT:]

    gsum_e = _make_gsum_f32(E, OUT, 10)
    gsum_n = _make_gsum_f32(N_PAD, OUT, 10)

    k1 = pl.pallas_call(
        _k1_body,
        grid=(E // _RB_E,),
        in_specs=[_row_spec(_RB_E, IN_DIM), _full_spec(IN_DIM, OUT),
                  _full_spec(1, OUT)],
        out_specs=_row_spec(_RB_E, OUT),
        out_shape=jax.ShapeDtypeStruct((E, OUT), jnp.float32),
    )
    k2 = pl.pallas_call(
        _k2_body,
        grid=(E // _RB_E,),
        in_specs=[_row_spec(_RB_E, OUT), _row_spec(_RB_E, OUT),
                  _full_spec(OUT, OUT), _full_spec(1, OUT)],
        out_specs=_row_spec(_RB_E, OUT),
        out_shape=jax.ShapeDtypeStruct((E, OUT), jnp.float32),
    )
    k3 = pl.pallas_call(
        _k3_body,
        grid=(E // _RB_E,),
        in_specs=[_row_spec(_RB_E, OUT), _row_spec(_RB_E, OUT),
                  _full_spec(OUT, OUT), _full_spec(1, OUT),
                  _full_spec(OUT, OUT), _full_spec(1, OUT)],
        out_specs=_row_spec(_RB_E, OUT),
        out_shape=jax.ShapeDtypeStruct((E, OUT), jnp.float32),
    )
    k4 = pl.pallas_call(
        _k4_body,
        grid=(N // _RB_N,),
        in_specs=[_row_spec(_RB_N, FEAT), _row_spec(_RB_N, OUT),
                  _full_spec(FEAT, OUT), _full_spec(OUT, OUT),
                  _full_spec(1, OUT)],
        out_specs=_row_spec(_RB_N, OUT),
        out_shape=jax.ShapeDtypeStruct((N, OUT), jnp.float32),
    )

    h0 = k1(edge_features, W_i, bi)
    nei1 = gsum_e(h0, bond_flat)
    h1 = k2(h0, nei1, W_h, bh)
    nei2 = gsum_e(h1, bond_flat)
    edge_hidden = k3(h0, nei2, W_h, bh, W_proj, bp)
    nm = gsum_n(edge_hidden, adj_flat)[:N]
    transformed = k4(node_features, nm, wt1, wt2, bt)
    return (transformed, edge_hidden)


# speedup vs baseline: 2.6237x; 2.6237x over previous
import functools

import jax
import jax.numpy as jnp
from jax import lax
from jax.experimental import pallas as pl
from jax.experimental.pallas import tpu as pltpu
from jax.experimental.pallas import tpu_sc as plsc

N = 10000
E = 160000
MAX_NB = 8
IN_DIM = 16
FEAT = 128
OUT = 128

NC, NS = 2, 16
NW = NC * NS
LANES = 16

N_PAD = 10496


def _make_gsum_f32(M, D, B):
    Mw = M // NW
    NB = Mw // B
    K = B * MAX_NB
    mesh = plsc.VectorSubcoreMesh(core_axis_name="c", subcore_axis_name="s",
                                  num_cores=NC, num_subcores=NS)

    assert NB % 2 == 1

    def body(table_hbm, idx_hbm, out_hbm, idx_v, rows_a, rows_b, out_v, sem):
        wid = lax.axis_index("s") * NC + lax.axis_index("c")
        row0 = wid * Mw
        pltpu.sync_copy(idx_hbm.at[pl.ds(row0 * MAX_NB, Mw * MAX_NB)], idx_v)

        def gather(b, buf):
            pltpu.async_copy(table_hbm.at[idx_v.at[pl.ds(b * K, K)]], buf, sem)

        def wait_rows(buf):
            pltpu.make_async_copy(table_hbm.at[pl.ds(0, K)], buf, sem).wait()

        def consume(b, buf):
            for i in range(B):
                for c in range(D // LANES):
                    sl = pl.ds(c * LANES, LANES)
                    acc = buf[MAX_NB * i, sl]
                    for j in range(1, MAX_NB):
                        acc = acc + buf[MAX_NB * i + j, sl]
                    out_v[i, sl] = acc
            pltpu.sync_copy(out_v, out_hbm.at[pl.ds(row0 + b * B, B)])

        gather(0, rows_a)

        def pair(p, carry):
            b = p * 2
            gather(b + 1, rows_b)
            wait_rows(rows_a)
            consume(b, rows_a)
            gather(b + 2, rows_a)
            wait_rows(rows_b)
            consume(b + 1, rows_b)
            return carry

        lax.fori_loop(0, (NB - 1) // 2, pair, 0)
        wait_rows(rows_a)
        consume(NB - 1, rows_a)

    return pl.kernel(
        body,
        out_type=jax.ShapeDtypeStruct((M, D), jnp.float32),
        mesh=mesh,
        scratch_types=[
            pltpu.VMEM((Mw * MAX_NB,), jnp.int32),
            pltpu.VMEM((K, D), jnp.float32),
            pltpu.VMEM((K, D), jnp.float32),
            pltpu.VMEM((B, D), jnp.float32),
            pltpu.SemaphoreType.DMA,
        ],
    )


def _dot(a, w):
    return jnp.dot(a, w, preferred_element_type=jnp.float32)


_RB_E = 2000
_RB_N = 2000


def _k1_body(ef, wi, bi, h0):
    h0[...] = jnp.maximum(_dot(ef[...], wi[...]) + bi[...], 0.0)


def _k2_body(h0, nei, wh, bh, h1):
    h1[...] = jnp.maximum(h0[...] + _dot(nei[...], wh[...]) + bh[...], 0.0)


def _k3_body(h0, nei, wh, bh, wp, bp, eh):
    t = jnp.maximum(h0[...] + _dot(nei[...], wh[...]) + bh[...], 0.0)
    eh[...] = _dot(t, wp[...]) + bp[...]


def _k4_body(nf, nm, wt1, wt2, bt, out):
    y = jnp.maximum(_dot(nf[...], wt1[...]) + _dot(nm[...], wt2[...]) + bt[...], 0.0)
    rows = lax.broadcasted_iota(jnp.int32, y.shape, 0)
    out[...] = jnp.where((pl.program_id(0) == 0) & (rows == 0), 0.0, y)


def _row_spec(rb, d):
    return pl.BlockSpec((rb, d), lambda i: (i, 0))


def _full_spec(r, c):
    return pl.BlockSpec((r, c), lambda i: (0, 0))


def kernel(node_features, edge_features, adj_list, bond_list,
           W_i, b_i, W_h, b_h, W_proj, b_proj, W_t, b_t):
    bond_flat = bond_list.reshape(-1)
    adj_flat = jnp.zeros((N_PAD * MAX_NB,), jnp.int32).at[: N * MAX_NB].set(
        adj_list.reshape(-1))
    bi = b_i.reshape(1, OUT)
    bh = b_h.reshape(1, OUT)
    bp = b_proj.reshape(1, OUT)
    bt = b_t.reshape(1, OUT)
    wt1 = W_t[:FEAT]
    wt2 = W_t[FEAT:]

    gsum_e = _make_gsum_f32(E, OUT, 8)
    gsum_n = _make_gsum_f32(N_PAD, OUT, 8)

    k1 = pl.pallas_call(
        _k1_body,
        grid=(E // _RB_E,),
        in_specs=[_row_spec(_RB_E, IN_DIM), _full_spec(IN_DIM, OUT),
                  _full_spec(1, OUT)],
        out_specs=_row_spec(_RB_E, OUT),
        out_shape=jax.ShapeDtypeStruct((E, OUT), jnp.float32),
    )
    k2 = pl.pallas_call(
        _k2_body,
        grid=(E // _RB_E,),
        in_specs=[_row_spec(_RB_E, OUT), _row_spec(_RB_E, OUT),
                  _full_spec(OUT, OUT), _full_spec(1, OUT)],
        out_specs=_row_spec(_RB_E, OUT),
        out_shape=jax.ShapeDtypeStruct((E, OUT), jnp.float32),
    )
    k3 = pl.pallas_call(
        _k3_body,
        grid=(E // _RB_E,),
        in_specs=[_row_spec(_RB_E, OUT), _row_spec(_RB_E, OUT),
                  _full_spec(OUT, OUT), _full_spec(1, OUT),
                  _full_spec(OUT, OUT), _full_spec(1, OUT)],
        out_specs=_row_spec(_RB_E, OUT),
        out_shape=jax.ShapeDtypeStruct((E, OUT), jnp.float32),
    )
    k4 = pl.pallas_call(
        _k4_body,
        grid=(N // _RB_N,),
        in_specs=[_row_spec(_RB_N, FEAT), _row_spec(_RB_N, OUT),
                  _full_spec(FEAT, OUT), _full_spec(OUT, OUT),
                  _full_spec(1, OUT)],
        out_specs=_row_spec(_RB_N, OUT),
        out_shape=jax.ShapeDtypeStruct((N, OUT), jnp.float32),
    )

    h0 = k1(edge_features, W_i, bi)
    nei1 = gsum_e(h0, bond_flat)
    h1 = k2(h0, nei1, W_h, bh)
    nei2 = gsum_e(h1, bond_flat)
    edge_hidden = k3(h0, nei2, W_h, bh, W_proj, bp)
    nm = gsum_n(edge_hidden, adj_flat)[:N]
    transformed = k4(node_features, nm, wt1, wt2, bt)
    return (transformed, edge_hidden)


# speedup vs baseline: 3.0782x; 1.1732x over previous
import functools

import jax
import jax.numpy as jnp
from jax import lax
from jax.experimental import pallas as pl
from jax.experimental.pallas import tpu as pltpu
from jax.experimental.pallas import tpu_sc as plsc

N = 10000
E = 160000
MAX_NB = 8
IN_DIM = 16
FEAT = 128
OUT = 128

NC, NS = 2, 16
NW = NC * NS
LANES = 16

N_PAD = 11520


def _make_gsum_f32(M, D, B):
    Mw = M // NW
    NB = Mw // B
    K = B * MAX_NB
    NSUB = 4
    KS = K // NSUB
    assert KS <= 128 and K % NSUB == 0 and KS % 8 == 0
    mesh = plsc.VectorSubcoreMesh(core_axis_name="c", subcore_axis_name="s",
                                  num_cores=NC, num_subcores=NS)

    assert NB % 2 == 1

    def body(table_hbm, idx_hbm, out_hbm, idx_v, rows_a, rows_b, out_v, sem):
        wid = lax.axis_index("s") * NC + lax.axis_index("c")
        row0 = wid * Mw
        pltpu.sync_copy(idx_hbm.at[pl.ds(row0 * MAX_NB, Mw * MAX_NB)], idx_v)

        def gather(b, buf):
            for q in range(NSUB):
                pltpu.async_copy(
                    table_hbm.at[idx_v.at[pl.ds(b * K + q * KS, KS)]],
                    buf.at[pl.ds(q * KS, KS)], sem)

        def wait_rows(buf):
            pltpu.make_async_copy(table_hbm.at[pl.ds(0, K)], buf, sem).wait()

        def consume(b, buf):
            def srow(i, carry):
                for c in range(D // LANES):
                    sl = pl.ds(c * LANES, LANES)
                    acc = buf[MAX_NB * i, sl]
                    for j in range(1, MAX_NB):
                        acc = acc + buf[MAX_NB * i + j, sl]
                    out_v[i, sl] = acc
                return carry

            lax.fori_loop(0, B, srow, 0)
            pltpu.sync_copy(out_v, out_hbm.at[pl.ds(row0 + b * B, B)])

        gather(0, rows_a)

        def pair(p, carry):
            b = p * 2
            gather(b + 1, rows_b)
            wait_rows(rows_a)
            consume(b, rows_a)
            gather(b + 2, rows_a)
            wait_rows(rows_b)
            consume(b + 1, rows_b)
            return carry

        lax.fori_loop(0, (NB - 1) // 2, pair, 0)
        wait_rows(rows_a)
        consume(NB - 1, rows_a)

    return pl.kernel(
        body,
        out_type=jax.ShapeDtypeStruct((M, D), jnp.float32),
        mesh=mesh,
        scratch_types=[
            pltpu.VMEM((Mw * MAX_NB,), jnp.int32),
            pltpu.VMEM((K, D), jnp.float32),
            pltpu.VMEM((K, D), jnp.float32),
            pltpu.VMEM((B, D), jnp.float32),
            pltpu.SemaphoreType.DMA,
        ],
    )


def _dot(a, w):
    return jnp.dot(a, w, preferred_element_type=jnp.float32)


_RB_E = 2000
_RB_N = 2000


def _k1_body(ef, wi, bi, h0):
    h0[...] = jnp.maximum(_dot(ef[...], wi[...]) + bi[...], 0.0)


def _k2_body(h0, nei, wh, bh, h1):
    h1[...] = jnp.maximum(h0[...] + _dot(nei[...], wh[...]) + bh[...], 0.0)


def _k3_body(h0, nei, wh, bh, wp, bp, eh):
    t = jnp.maximum(h0[...] + _dot(nei[...], wh[...]) + bh[...], 0.0)
    eh[...] = _dot(t, wp[...]) + bp[...]


def _k4_body(nf, nm, wt1, wt2, bt, out):
    y = jnp.maximum(_dot(nf[...], wt1[...]) + _dot(nm[...], wt2[...]) + bt[...], 0.0)
    rows = lax.broadcasted_iota(jnp.int32, y.shape, 0)
    out[...] = jnp.where((pl.program_id(0) == 0) & (rows == 0), 0.0, y)


def _row_spec(rb, d):
    return pl.BlockSpec((rb, d), lambda i: (i, 0))


def _full_spec(r, c):
    return pl.BlockSpec((r, c), lambda i: (0, 0))


def kernel(node_features, edge_features, adj_list, bond_list,
           W_i, b_i, W_h, b_h, W_proj, b_proj, W_t, b_t):
    bond_flat = bond_list.reshape(-1)
    adj_flat = jnp.zeros((N_PAD * MAX_NB,), jnp.int32).at[: N * MAX_NB].set(
        adj_list.reshape(-1))
    bi = b_i.reshape(1, OUT)
    bh = b_h.reshape(1, OUT)
    bp = b_proj.reshape(1, OUT)
    bt = b_t.reshape(1, OUT)
    wt1 = W_t[:FEAT]
    wt2 = W_t[FEAT:]

    gsum_e = _make_gsum_f32(E, OUT, 40)
    gsum_n = _make_gsum_f32(N_PAD, OUT, 40)

    k1 = pl.pallas_call(
        _k1_body,
        grid=(E // _RB_E,),
        in_specs=[_row_spec(_RB_E, IN_DIM), _full_spec(IN_DIM, OUT),
                  _full_spec(1, OUT)],
        out_specs=_row_spec(_RB_E, OUT),
        out_shape=jax.ShapeDtypeStruct((E, OUT), jnp.float32),
    )
    k2 = pl.pallas_call(
        _k2_body,
        grid=(E // _RB_E,),
        in_specs=[_row_spec(_RB_E, OUT), _row_spec(_RB_E, OUT),
                  _full_spec(OUT, OUT), _full_spec(1, OUT)],
        out_specs=_row_spec(_RB_E, OUT),
        out_shape=jax.ShapeDtypeStruct((E, OUT), jnp.float32),
    )
    k3 = pl.pallas_call(
        _k3_body,
        grid=(E // _RB_E,),
        in_specs=[_row_spec(_RB_E, OUT), _row_spec(_RB_E, OUT),
                  _full_spec(OUT, OUT), _full_spec(1, OUT),
                  _full_spec(OUT, OUT), _full_spec(1, OUT)],
        out_specs=_row_spec(_RB_E, OUT),
        out_shape=jax.ShapeDtypeStruct((E, OUT), jnp.float32),
    )
    k4 = pl.pallas_call(
        _k4_body,
        grid=(N // _RB_N,),
        in_specs=[_row_spec(_RB_N, FEAT), _row_spec(_RB_N, OUT),
                  _full_spec(FEAT, OUT), _full_spec(OUT, OUT),
                  _full_spec(1, OUT)],
        out_specs=_row_spec(_RB_N, OUT),
        out_shape=jax.ShapeDtypeStruct((N, OUT), jnp.float32),
    )

    h0 = k1(edge_features, W_i, bi)
    nei1 = gsum_e(h0, bond_flat)
    h1 = k2(h0, nei1, W_h, bh)
    nei2 = gsum_e(h1, bond_flat)
    edge_hidden = k3(h0, nei2, W_h, bh, W_proj, bp)
    nm = gsum_n(edge_hidden, adj_flat)[:N]
    transformed = k4(node_features, nm, wt1, wt2, bt)
    return (transformed, edge_hidden)


# speedup vs baseline: 3.6074x; 1.1719x over previous
import jax
import jax.numpy as jnp
from jax import lax
from jax.experimental import pallas as pl
from jax.experimental.pallas import tpu as pltpu
from jax.experimental.pallas import tpu_sc as plsc

N = 10000
E = 160000
MAX_NB = 8
IN_DIM = 16
FEAT = 128
OUT = 128

NC, NS = 2, 16
NW = NC * NS
LANES = 16

N_PAD = 10496


def _make_gsum(M, B):
    Mw = M // NW
    NB = Mw // B
    K = B * MAX_NB
    NSUB = 4
    KS = K // NSUB
    assert KS <= 128 and K % NSUB == 0 and KS % 8 == 0
    assert NB % 2 == 1
    mesh = plsc.VectorSubcoreMesh(core_axis_name="c", subcore_axis_name="s",
                                  num_cores=NC, num_subcores=NS)

    def body(table_hbm, idx_hbm, out_hbm, idx_v, rows_a, rows_b, out_v, sem):
        wid = lax.axis_index("s") * NC + lax.axis_index("c")
        row0 = wid * Mw
        pltpu.sync_copy(idx_hbm.at[pl.ds(row0 * MAX_NB, Mw * MAX_NB)], idx_v)

        def gather(b, buf):
            for q in range(NSUB):
                pltpu.async_copy(
                    table_hbm.at[idx_v.at[pl.ds(b * K + q * KS, KS)]],
                    buf.at[pl.ds(q * KS, KS)], sem)

        def wait_rows(buf):
            pltpu.make_async_copy(table_hbm.at[pl.ds(0, K)], buf, sem).wait()

        def consume(b, buf):
            def srow(i, carry):
                for c in range(OUT // LANES):
                    sl = pl.ds(c * LANES, LANES)
                    acc = buf[MAX_NB * i, sl]
                    for j in range(1, MAX_NB):
                        acc = acc + buf[MAX_NB * i + j, sl]
                    out_v[i, sl] = acc
                return carry

            lax.fori_loop(0, B, srow, 0)
            pltpu.sync_copy(out_v, out_hbm.at[pl.ds(row0 + b * B, B)])

        gather(0, rows_a)

        def pair(p, carry):
            b = p * 2
            wait_rows(rows_a)
            gather(b + 1, rows_b)
            consume(b, rows_a)
            wait_rows(rows_b)
            gather(b + 2, rows_a)
            consume(b + 1, rows_b)
            return carry

        lax.fori_loop(0, (NB - 1) // 2, pair, 0)
        wait_rows(rows_a)
        consume(NB - 1, rows_a)

    return pl.kernel(
        body,
        out_type=jax.ShapeDtypeStruct((M, OUT), jnp.float32),
        mesh=mesh,
        scratch_types=[
            pltpu.VMEM((Mw * MAX_NB,), jnp.int32),
            pltpu.VMEM((K, OUT), jnp.float32),
            pltpu.VMEM((K, OUT), jnp.float32),
            pltpu.VMEM((B, OUT), jnp.float32),
            pltpu.SemaphoreType.DMA,
        ],
    )


def _dot(a, w):
    return jnp.dot(a, w, preferred_element_type=jnp.float32)


_RB_E = 2000
_RB_N = 2000


def _k1_body(ef, wi, bi, h0):
    h0[...] = jnp.maximum(_dot(ef[...], wi[...]) + bi[...], 0.0)


def _k2_body(h0, nei, wh, bh, h1):
    h1[...] = jnp.maximum(h0[...] + _dot(nei[...], wh[...]) + bh[...], 0.0)


def _k3_body(h0, nei, wh, bh, wp, bp, eh):
    t = jnp.maximum(h0[...] + _dot(nei[...], wh[...]) + bh[...], 0.0)
    eh[...] = _dot(t, wp[...]) + bp[...]


def _k4_body(nf, nm, wt1, wt2, bt, out):
    y = jnp.maximum(_dot(nf[...], wt1[...]) + _dot(nm[...], wt2[...]) + bt[...], 0.0)
    rows = lax.broadcasted_iota(jnp.int32, y.shape, 0)
    out[...] = jnp.where((pl.program_id(0) == 0) & (rows == 0), 0.0, y)


def _row_spec(rb, d):
    return pl.BlockSpec((rb, d), lambda i: (i, 0))


def _full_spec(r, c):
    return pl.BlockSpec((r, c), lambda i: (0, 0))


def kernel(node_features, edge_features, adj_list, bond_list,
           W_i, b_i, W_h, b_h, W_proj, b_proj, W_t, b_t):
    bond_flat = bond_list.reshape(-1)
    adj_flat = jnp.zeros((N_PAD * MAX_NB,), jnp.int32).at[: N * MAX_NB].set(
        adj_list.reshape(-1))
    bi = b_i.reshape(1, OUT)
    bh = b_h.reshape(1, OUT)
    bp = b_proj.reshape(1, OUT)
    bt = b_t.reshape(1, OUT)
    wt1 = W_t[:FEAT]
    wt2 = W_t[FEAT:]

    gsum_e = _make_gsum(E, 40)
    gsum_n = _make_gsum(N_PAD, 8)

    k1 = pl.pallas_call(
        _k1_body,
        grid=(E // _RB_E,),
        in_specs=[_row_spec(_RB_E, IN_DIM), _full_spec(IN_DIM, OUT),
                  _full_spec(1, OUT)],
        out_specs=_row_spec(_RB_E, OUT),
        out_shape=jax.ShapeDtypeStruct((E, OUT), jnp.float32),
    )
    k2 = pl.pallas_call(
        _k2_body,
        grid=(E // _RB_E,),
        in_specs=[_row_spec(_RB_E, OUT), _row_spec(_RB_E, OUT),
                  _full_spec(OUT, OUT), _full_spec(1, OUT)],
        out_specs=_row_spec(_RB_E, OUT),
        out_shape=jax.ShapeDtypeStruct((E, OUT), jnp.float32),
    )
    k3 = pl.pallas_call(
        _k3_body,
        grid=(E // _RB_E,),
        in_specs=[_row_spec(_RB_E, OUT), _row_spec(_RB_E, OUT),
                  _full_spec(OUT, OUT), _full_spec(1, OUT),
                  _full_spec(OUT, OUT), _full_spec(1, OUT)],
        out_specs=_row_spec(_RB_E, OUT),
        out_shape=jax.ShapeDtypeStruct((E, OUT), jnp.float32),
    )
    k4 = pl.pallas_call(
        _k4_body,
        grid=(N // _RB_N,),
        in_specs=[_row_spec(_RB_N, FEAT), _row_spec(_RB_N, OUT),
                  _full_spec(FEAT, OUT), _full_spec(OUT, OUT),
                  _full_spec(1, OUT)],
        out_specs=_row_spec(_RB_N, OUT),
        out_shape=jax.ShapeDtypeStruct((N, OUT), jnp.float32),
    )

    h0 = k1(edge_features, W_i, bi)
    nei1 = gsum_e(h0, bond_flat)
    h1 = k2(h0, nei1, W_h, bh)
    nei2 = gsum_e(h1, bond_flat)
    edge_hidden = k3(h0, nei2, W_h, bh, W_proj, bp)
    nm = gsum_n(edge_hidden, adj_flat)[:N]
    transformed = k4(node_features, nm, wt1, wt2, bt)
    return (transformed, edge_hidden)
